# Initial kernel scaffold; baseline (speedup 1.0000x reference)
import jax
import jax.numpy as jnp
from jax.experimental import pallas as pl


def kernel(x, w1, gamma1, beta1, rm1, rv1, w2, gamma2, beta2, rm2, rv2):
    raise NotImplementedError("write your pallas kernel here")


# speedup vs baseline: 1.1185x; 1.1185x over previous
import functools

import jax
import jax.numpy as jnp
from jax.experimental import pallas as pl
from jax.experimental.pallas import tpu as pltpu

_BN_EPS = 1e-5
_VMEM_LIMIT_BYTES = 64 * 1024 * 1024


def _se_fused_kernel(inv_s, s_total, needs_mask,
                     x_ref, w1_ref, b1_ref, w2_ref, b2_ref, o_ref):
    x = x_ref[0].astype(jnp.float32)
    if needs_mask:
        lane = jax.lax.broadcasted_iota(jnp.int32, x.shape, 1)
        xs = jnp.where(lane < s_total, x, 0.0)
    else:
        xs = x
    m = jnp.sum(xs, axis=1, keepdims=True) * inv_s
    y1 = jnp.dot(w1_ref[...], m,
                 preferred_element_type=jnp.float32) + b1_ref[...]
    z = jnp.dot(w2_ref[...], y1,
                preferred_element_type=jnp.float32) + b2_ref[...]
    scale = jax.nn.sigmoid(z) + 0.5
    o_ref[0] = (x * scale).astype(o_ref.dtype)


def kernel(x, w1, gamma1, beta1, rm1, rv1, w2, gamma2, beta2, rm2, rv2):
    b, c, t, h, w_ = x.shape
    s = t * h * w_
    x_flat = x.reshape(b, c, s)
    cr = w1.shape[0]

    f32 = jnp.float32
    a1 = gamma1.astype(f32) * jax.lax.rsqrt(rv1.astype(f32) + _BN_EPS)
    w1f = w1.astype(f32) * a1[:, None]
    b1f = (beta1.astype(f32) - rm1.astype(f32) * a1)[:, None]
    a2 = gamma2.astype(f32) * jax.lax.rsqrt(rv2.astype(f32) + _BN_EPS)
    w2f = w2.astype(f32) * a2[:, None]
    b2f = (beta2.astype(f32) - rm2.astype(f32) * a2)[:, None]

    needs_mask = (s % 128) != 0

    out_flat = pl.pallas_call(
        functools.partial(_se_fused_kernel, 1.0 / float(s), s, needs_mask),
        out_shape=jax.ShapeDtypeStruct((b, c, s), x.dtype),
        grid=(b,),
        in_specs=[
            pl.BlockSpec((1, c, s), lambda bi: (bi, 0, 0)),
            pl.BlockSpec((cr, c), lambda bi: (0, 0)),
            pl.BlockSpec((cr, 1), lambda bi: (0, 0)),
            pl.BlockSpec((c, cr), lambda bi: (0, 0)),
            pl.BlockSpec((c, 1), lambda bi: (0, 0)),
        ],
        out_specs=pl.BlockSpec((1, c, s), lambda bi: (bi, 0, 0)),
        compiler_params=pltpu.CompilerParams(
            dimension_semantics=("parallel",),
            vmem_limit_bytes=_VMEM_LIMIT_BYTES),
    )(x_flat, w1f, b1f, w2f, b2f)

    return out_flat.reshape(b, c, t, h, w_)


# speedup vs baseline: 6.3340x; 5.6632x over previous
import functools

import jax
import jax.numpy as jnp
from jax.experimental import pallas as pl
from jax.experimental.pallas import tpu as pltpu

_BN_EPS = 1e-5
_VMEM_LIMIT_BYTES = 64 * 1024 * 1024


def _se_fused_kernel(inv_s, s_total, needs_mask,
                     x_ref, w1_ref, b1_ref, w2_ref, b2_ref, o_ref):
    x = x_ref[0].astype(jnp.float32)
    if needs_mask:
        row = jax.lax.broadcasted_iota(jnp.int32, x.shape, 0)
        xs = jnp.where(row < s_total, x, 0.0)
    else:
        xs = x
    m = jnp.sum(xs, axis=0, keepdims=True) * inv_s
    y1 = jnp.dot(m, w1_ref[...],
                 preferred_element_type=jnp.float32) + b1_ref[...]
    z = jnp.dot(y1, w2_ref[...],
                preferred_element_type=jnp.float32) + b2_ref[...]
    scale = jax.nn.sigmoid(z) + 0.5
    o_ref[0] = (x * scale).astype(o_ref.dtype)


def kernel(x, w1, gamma1, beta1, rm1, rv1, w2, gamma2, beta2, rm2, rv2):
    b, c, t, h, w_ = x.shape
    s = t * h * w_
    cr = w1.shape[0]

    f32 = jnp.float32
    a1 = gamma1.astype(f32) * jax.lax.rsqrt(rv1.astype(f32) + _BN_EPS)
    w1f = (w1.astype(f32) * a1[:, None]).T
    b1f = (beta1.astype(f32) - rm1.astype(f32) * a1)[None, :]
    a2 = gamma2.astype(f32) * jax.lax.rsqrt(rv2.astype(f32) + _BN_EPS)
    w2f = (w2.astype(f32) * a2[:, None]).T
    b2f = (beta2.astype(f32) - rm2.astype(f32) * a2)[None, :]

    xt = jnp.transpose(x, (0, 3, 4, 2, 1))
    x_flat = xt.reshape(b, s, c)

    needs_mask = (s % 8) != 0

    out_flat = pl.pallas_call(
        functools.partial(_se_fused_kernel, 1.0 / float(s), s, needs_mask),
        out_shape=jax.ShapeDtypeStruct((b, s, c), x.dtype),
        grid=(b,),
        in_specs=[
            pl.BlockSpec((1, s, c), lambda bi: (bi, 0, 0)),
            pl.BlockSpec((c, cr), lambda bi: (0, 0)),
            pl.BlockSpec((1, cr), lambda bi: (0, 0)),
            pl.BlockSpec((cr, c), lambda bi: (0, 0)),
            pl.BlockSpec((1, c), lambda bi: (0, 0)),
        ],
        out_specs=pl.BlockSpec((1, s, c), lambda bi: (bi, 0, 0)),
        compiler_params=pltpu.CompilerParams(
            dimension_semantics=("parallel",),
            vmem_limit_bytes=_VMEM_LIMIT_BYTES),
    )(x_flat, w1f, b1f, w2f, b2f)

    out = out_flat.reshape(b, h, w_, t, c)
    return jnp.transpose(out, (0, 4, 3, 1, 2))


# speedup vs baseline: 7.0680x; 1.1159x over previous
import functools

import jax
import jax.numpy as jnp
from jax.experimental import pallas as pl
from jax.experimental.pallas import tpu as pltpu

_BN_EPS = 1e-5
_VMEM_LIMIT_BYTES = 64 * 1024 * 1024


def _se_fused_kernel(inv_s, s_total, needs_mask,
                     x_ref, w1_ref, w2_ref, g1_ref, be1_ref, rm1_ref, rv1_ref,
                     g2_ref, be2_ref, rm2_ref, rv2_ref, o_ref):
    x = x_ref[0]
    if needs_mask:
        row = jax.lax.broadcasted_iota(jnp.int32, x.shape, 0)
        xs = jnp.where(row < s_total, x, 0.0)
    else:
        xs = x
    m = jnp.sum(xs, axis=0, keepdims=True) * inv_s

    a1 = g1_ref[...] * jax.lax.rsqrt(rv1_ref[...] + _BN_EPS)
    a2 = g2_ref[...] * jax.lax.rsqrt(rv2_ref[...] + _BN_EPS)
    t1 = jax.lax.dot_general(m, w1_ref[...], (((1,), (1,)), ((), ())),
                             preferred_element_type=jnp.float32)
    y1 = a1 * t1 + be1_ref[...] - rm1_ref[...] * a1
    t2 = jax.lax.dot_general(y1, w2_ref[...], (((1,), (1,)), ((), ())),
                             preferred_element_type=jnp.float32)
    z = a2 * t2 + be2_ref[...] - rm2_ref[...] * a2
    scale = jax.nn.sigmoid(z) + 0.5
    o_ref[0] = x * scale


def kernel(x, w1, gamma1, beta1, rm1, rv1, w2, gamma2, beta2, rm2, rv2):
    b, c, t, h, w_ = x.shape
    s = t * h * w_
    cr = w1.shape[0]

    xt = jnp.transpose(x, (0, 3, 4, 2, 1))
    x_flat = xt.reshape(b, s, c)

    needs_mask = (s % 8) != 0

    row = lambda v: v.reshape(1, v.shape[0]).astype(jnp.float32)
    vec_spec = lambda n: pl.BlockSpec((1, n), lambda bi: (0, 0))

    out_flat = pl.pallas_call(
        functools.partial(_se_fused_kernel, 1.0 / float(s), s, needs_mask),
        out_shape=jax.ShapeDtypeStruct((b, s, c), x.dtype),
        grid=(b,),
        in_specs=[
            pl.BlockSpec((1, s, c), lambda bi: (bi, 0, 0)),
            pl.BlockSpec((cr, c), lambda bi: (0, 0)),
            pl.BlockSpec((c, cr), lambda bi: (0, 0)),
            vec_spec(cr), vec_spec(cr), vec_spec(cr), vec_spec(cr),
            vec_spec(c), vec_spec(c), vec_spec(c), vec_spec(c),
        ],
        out_specs=pl.BlockSpec((1, s, c), lambda bi: (bi, 0, 0)),
        compiler_params=pltpu.CompilerParams(
            dimension_semantics=("parallel",),
            vmem_limit_bytes=_VMEM_LIMIT_BYTES),
    )(x_flat, w1, w2,
      row(gamma1), row(beta1), row(rm1), row(rv1),
      row(gamma2), row(beta2), row(rm2), row(rv2))

    out = out_flat.reshape(b, h, w_, t, c)
    return jnp.transpose(out, (0, 4, 3, 1, 2))


# speedup vs baseline: 7.3685x; 1.0425x over previous
import functools

import jax
import jax.numpy as jnp
from jax.experimental import pallas as pl
from jax.experimental.pallas import tpu as pltpu

_BN_EPS = 1e-5
_VMEM_LIMIT_BYTES = 64 * 1024 * 1024


def _se_fused_kernel(inv_s, s_total, needs_mask,
                     x_ref, w1_ref, w2_ref, g1_ref, be1_ref, rm1_ref, rv1_ref,
                     g2_ref, be2_ref, rm2_ref, rv2_ref, o_ref):
    x = x_ref[0]
    if needs_mask:
        row = jax.lax.broadcasted_iota(jnp.int32, x.shape, 0)
        xs = jnp.where(row < s_total, x, 0.0)
    else:
        xs = x
    m = jnp.sum(xs, axis=0, keepdims=True) * inv_s

    a1 = g1_ref[...] * jax.lax.rsqrt(rv1_ref[...] + _BN_EPS)
    a2 = g2_ref[...] * jax.lax.rsqrt(rv2_ref[...] + _BN_EPS)
    t1 = jax.lax.dot_general(m, w1_ref[...], (((1,), (1,)), ((), ())),
                             preferred_element_type=jnp.float32)
    y1 = a1 * t1 + be1_ref[...] - rm1_ref[...] * a1
    t2 = jnp.dot(y1, w2_ref[...],
                 preferred_element_type=jnp.float32)
    z = a2 * t2 + be2_ref[...] - rm2_ref[...] * a2
    scale = jax.nn.sigmoid(z) + 0.5
    o_ref[0] = x * scale


def kernel(x, w1, gamma1, beta1, rm1, rv1, w2, gamma2, beta2, rm2, rv2):
    b, c, t, h, w_ = x.shape
    s = t * h * w_
    cr = w1.shape[0]

    xt = jnp.transpose(x, (0, 3, 4, 2, 1))
    x_flat = xt.reshape(b, s, c)

    needs_mask = (s % 8) != 0

    row = lambda v: v.reshape(1, v.shape[0]).astype(jnp.float32)
    vec_spec = lambda n: pl.BlockSpec((1, n), lambda bi: (0, 0))

    out_flat = pl.pallas_call(
        functools.partial(_se_fused_kernel, 1.0 / float(s), s, needs_mask),
        out_shape=jax.ShapeDtypeStruct((b, s, c), x.dtype),
        grid=(b,),
        in_specs=[
            pl.BlockSpec((1, s, c), lambda bi: (bi, 0, 0)),
            pl.BlockSpec((cr, c), lambda bi: (0, 0)),
            pl.BlockSpec((cr, c), lambda bi: (0, 0)),
            vec_spec(cr), vec_spec(cr), vec_spec(cr), vec_spec(cr),
            vec_spec(c), vec_spec(c), vec_spec(c), vec_spec(c),
        ],
        out_specs=pl.BlockSpec((1, s, c), lambda bi: (bi, 0, 0)),
        compiler_params=pltpu.CompilerParams(
            dimension_semantics=("parallel",),
            vmem_limit_bytes=_VMEM_LIMIT_BYTES),
    )(x_flat, w1, w2.T,
      row(gamma1), row(beta1), row(rm1), row(rv1),
      row(gamma2), row(beta2), row(rm2), row(rv2))

    out = out_flat.reshape(b, h, w_, t, c)
    return jnp.transpose(out, (0, 4, 3, 1, 2))
